# baseline (speedup 1.0000x reference)
import functools

import jax
import jax.numpy as jnp
from jax import lax
from jax.experimental import pallas as pl
from jax.experimental.pallas import tpu as pltpu
from jax.experimental.pallas import tpu_sc as plsc

_W = 32
_NB = 2

_CP = pltpu.CompilerParams(needs_layout_passes=False)


def _make_transpose_kernel(emb_dim, vocab):
    mesh = plsc.VectorSubcoreMesh(core_axis_name="c", subcore_axis_name="s")
    num_cores = mesh.num_cores
    n_full = vocab // 128
    per_w = n_full // _W
    n_tail = n_full - per_w * _W

    @functools.partial(
        pl.kernel,
        out_type=jax.ShapeDtypeStruct((vocab // 2, 2 * emb_dim), jnp.float32),
        mesh=mesh,
        scratch_types=[
            [pltpu.VMEM((emb_dim, 128), jnp.float32) for _ in range(_NB)],
            [pltpu.VMEM((64, 2 * emb_dim), jnp.float32) for _ in range(_NB)],
            [pltpu.SemaphoreType.DMA for _ in range(_NB)],
            [pltpu.SemaphoreType.DMA for _ in range(_NB)],
        ],
        compiler_params=_CP,
    )
    def tk(tt_hbm, tail_hbm, out_hbm, buf, tbuf, gs, ps):
        wid = lax.axis_index("s") * num_cores + lax.axis_index("c")
        iota16 = lax.iota(jnp.int32, 16)
        rows = [iota16 + (dg % 4) * 16 for dg in range(8)]

        def chunk_of(t):
            return wid + t * _W

        def fire(t, b):
            return pltpu.async_copy(
                tt_hbm.at[:, pl.ds(chunk_of(t) * 128, 128)], buf[b], gs[b]
            )

        def wait_gather(b):
            pltpu.make_async_copy(
                tt_hbm.at[:, pl.ds(0, 128)], buf[b], gs[b]
            ).wait()

        def transpose(b):
            def kloop(k, carry):
                c0 = jnp.broadcast_to(2 * k, (16,)).astype(jnp.int32)
                c1 = c0 + 1
                for dg in range(8):
                    cols = c0 if dg < 4 else c1
                    tbuf[b][k, pl.ds(dg * 16, 16)] = plsc.load_gather(
                        buf[b], [rows[dg], cols]
                    )
                return carry

            lax.fori_loop(0, 64, kloop, 0, unroll=2)

        def put(t, b):
            return pltpu.async_copy(
                tbuf[b], out_hbm.at[pl.ds(chunk_of(t) * 64, 64)], ps[b]
            )

        def wait_put(b):
            pltpu.make_async_copy(
                tbuf[b], out_hbm.at[pl.ds(0, 64)], ps[b]
            ).wait()

        for b in range(_NB):
            fire(b, b)

        def group(g, carry):
            for b in range(_NB):
                t = g * _NB + b
                wait_gather(b)
                transpose(b)
                @pl.when(g > 0)
                def _():
                    wait_put(b)
                put(t, b)
                fire(t + _NB, b)
            return carry

        n_groups = per_w // _NB
        lax.fori_loop(0, n_groups - 1, group, 0, unroll=False)

        for b in range(_NB):
            t = (per_w - _NB) + b
            wait_gather(b)
            transpose(b)
            wait_put(b)
            put(t, b)
        for b in range(_NB):
            wait_put(b)

        @pl.when(wid < n_tail)
        def _():
            c = per_w * _W + wid
            pltpu.async_copy(
                tt_hbm.at[:, pl.ds(c * 128, 128)], buf[0], gs[0]
            ).wait()
            transpose(0)
            if n_tail > 0:
                @pl.when(wid < n_tail - 1)
                def _():
                    pltpu.async_copy(
                        tbuf[0], out_hbm.at[pl.ds(c * 64, 64)], ps[0]
                    ).wait()

                @pl.when(wid == n_tail - 1)
                def _():
                    pltpu.async_copy(
                        tbuf[0].at[pl.ds(0, 32)],
                        out_hbm.at[pl.ds(c * 64, 32)],
                        ps[0],
                    ).wait()

        @pl.when(wid == n_tail)
        def _():
            pltpu.sync_copy(tail_hbm, tbuf[0])
            pltpu.async_copy(
                tbuf[0], out_hbm.at[pl.ds(vocab // 2 - 64, 64)], ps[0]
            ).wait()

    return tk


def _make_gather_kernel(seq, batch, emb_dim, vocab):
    mesh = plsc.VectorSubcoreMesh(core_axis_name="c", subcore_axis_name="s")
    num_cores = mesh.num_cores
    cb = batch // _W

    @functools.partial(
        pl.kernel,
        out_type=jax.ShapeDtypeStruct((seq, emb_dim, batch), jnp.float32),
        mesh=mesh,
        scratch_types=[
            pltpu.VMEM((seq, cb), jnp.int32),
            [pltpu.VMEM((cb,), jnp.int32) for _ in range(_NB)],
            [pltpu.VMEM((cb,), jnp.int32) for _ in range(_NB)],
            [pltpu.VMEM((cb, 2 * emb_dim), jnp.float32) for _ in range(_NB)],
            [pltpu.VMEM((emb_dim, cb), jnp.float32) for _ in range(_NB)],
            [pltpu.SemaphoreType.DMA for _ in range(_NB)],
            [pltpu.SemaphoreType.DMA for _ in range(_NB)],
        ],
        compiler_params=_CP,
    )
    def gk(xt_hbm, tbl_hbm, out_hbm, idx_v, vrow, hb, buf, tbuf, gs, ps):
        wid = lax.axis_index("s") * num_cores + lax.axis_index("c")
        c0 = wid * cb
        iota16 = lax.iota(jnp.int32, 16)
        rows = [iota16 + ig * 16 for ig in range(cb // 16)]

        pltpu.sync_copy(xt_hbm.at[:, pl.ds(c0, cb)], idx_v)

        def prep_fire(j, b):
            for ig in range(cb // 16):
                v16 = idx_v[j, pl.ds(ig * 16, 16)]
                vrow[b][pl.ds(ig * 16, 16)] = lax.shift_right_logical(v16, 1)
                hb[b][pl.ds(ig * 16, 16)] = lax.bitwise_and(v16, 1) * emb_dim
            return pltpu.async_copy(tbl_hbm.at[vrow[b]], buf[b], gs[b])

        def wait_gather(b):
            pltpu.make_async_copy(tbl_hbm.at[vrow[b]], buf[b], gs[b]).wait()

        def transpose(b):
            hbs = tuple(hb[b][pl.ds(ig * 16, 16)] for ig in range(cb // 16))

            def dloop(d, carry):
                for ig in range(cb // 16):
                    tbuf[b][d, pl.ds(ig * 16, 16)] = plsc.load_gather(
                        buf[b], [rows[ig], carry[ig] + d]
                    )
                return carry

            lax.fori_loop(0, emb_dim, dloop, hbs, unroll=2)

        def put(j, b):
            return pltpu.async_copy(
                tbuf[b], out_hbm.at[j, :, pl.ds(c0, cb)], ps[b]
            )

        def wait_put(b):
            pltpu.make_async_copy(
                tbuf[b], out_hbm.at[0, :, pl.ds(c0, cb)], ps[b]
            ).wait()

        for b in range(_NB):
            prep_fire(b, b)

        def group(g, carry):
            for b in range(_NB):
                j = g * _NB + b
                wait_gather(b)
                transpose(b)
                @pl.when(g > 0)
                def _():
                    wait_put(b)
                put(j, b)
                prep_fire(j + _NB, b)
            return carry

        n_groups = seq // _NB
        lax.fori_loop(0, n_groups - 1, group, 0, unroll=False)

        for b in range(_NB):
            j = (seq - _NB) + b
            wait_gather(b)
            transpose(b)
            wait_put(b)
            put(j, b)
        for b in range(_NB):
            wait_put(b)

    return gk


def kernel(x, table):
    b0, b1 = x.shape
    vocab, emb_dim = table.shape
    xt = x.T.astype(jnp.int32)
    tt = table.T
    tail = table[vocab - 128:].reshape(64, 2 * emb_dim)
    tk = _make_transpose_kernel(emb_dim, vocab)
    tbl = tk(tt, tail)
    gk = _make_gather_kernel(b1, b0, emb_dim, vocab)
    out = gk(xt, tbl)
    return out.transpose(2, 0, 1)


# speedup vs baseline: 2.2190x; 2.2190x over previous
import functools

import jax
import jax.numpy as jnp
from jax import lax
from jax.experimental import pallas as pl
from jax.experimental.pallas import tpu as pltpu
from jax.experimental.pallas import tpu_sc as plsc

_C = 128
_NBUF = 4


def _make_emb_kernel(n_chunks, chunks_per_w, emb_dim):
    mesh = plsc.VectorSubcoreMesh(core_axis_name="c", subcore_axis_name="s")
    num_cores = mesh.num_cores

    @functools.partial(
        pl.kernel,
        out_type=jax.ShapeDtypeStruct((n_chunks * _C, emb_dim), jnp.float32),
        mesh=mesh,
        scratch_types=[
            pltpu.VMEM((chunks_per_w, _C), jnp.int32),
            [pltpu.VMEM((_C, emb_dim), jnp.float32) for _ in range(_NBUF)],
            [pltpu.SemaphoreType.DMA for _ in range(_NBUF)],
        ],
        compiler_params=pltpu.CompilerParams(use_tc_tiling_on_sc=False),
    )
    def emb(x_hbm, table_hbm, out_hbm, idx_v, rows, sems):
        wid = lax.axis_index("s") * num_cores + lax.axis_index("c")
        chunk0 = wid * chunks_per_w
        pltpu.sync_copy(x_hbm.at[pl.ds(chunk0, chunks_per_w)], idx_v)

        def gather_chunk(i, b):
            return pltpu.async_copy(table_hbm.at[idx_v.at[i]], rows[b], sems[b])

        def drain_chunk(i, b):
            pltpu.make_async_copy(
                table_hbm.at[idx_v.at[i]], rows[b], sems[b]
            ).wait()
            pltpu.sync_copy(
                rows[b], out_hbm.at[pl.ds((chunk0 + i) * _C, _C)]
            )

        for b in range(_NBUF):
            gather_chunk(b, b)

        def body(k, carry):
            for b in range(_NBUF):
                i = k * _NBUF + b
                drain_chunk(i, b)
                gather_chunk(i + _NBUF, b)
            return carry

        n_full = chunks_per_w // _NBUF - 1
        lax.fori_loop(0, n_full, body, 0, unroll=False)

        for b in range(_NBUF):
            drain_chunk(n_full * _NBUF + b, b)

    return emb


def kernel(x, table):
    b0, b1 = x.shape
    vocab, emb_dim = table.shape
    n = b0 * b1
    n_chunks = n // _C
    n_workers = 32
    chunks_per_w = n_chunks // n_workers
    xf = x.reshape(n_chunks, _C).astype(jnp.int32)
    emb = _make_emb_kernel(n_chunks, chunks_per_w, emb_dim)
    out = emb(xf, table)
    return out.reshape(b0, b1, emb_dim)
